# baseline (speedup 1.0000x reference)
import jax
import jax.numpy as jnp
from jax import lax
from jax.experimental import pallas as pl
from jax.experimental.pallas import tpu as pltpu
from jax.experimental.pallas import tpu_sc as plsc

B, C, H, W = 2, 96, 384, 384
VOX = 128 * 128 * 16
NPIX = H * W
NCH = B * C
NC, NS = 2, 16
CH_PER_SC = NCH // NC
PIX_PER_TILE = NPIX // NS
VOX_PER_TILE = VOX // NS


def _sc_scatter(feat_hbm, idx_hbm, out_hbm,
                v0, v1, v2, idx1d, zeros, g0, g1, g2,
                sv0, sv1, sv2, swb0, swb1, swb2, sz0, sz1, sz2):
    c = lax.axis_index("c")
    s = lax.axis_index("s")
    rows = pl.ds(s * PIX_PER_TILE, PIX_PER_TILE)
    ch_lo = c * CH_PER_SC
    ch_hi = ch_lo + CH_PER_SC - 1
    x0 = s * VOX_PER_TILE
    vslice = pl.ds(x0, VOX_PER_TILE)

    vbufs = (v0, v1, v2)
    vsems = (sv0, sv1, sv2)
    grids = (g0, g1, g2)
    wsems = (swb0, swb1, swb2)
    zsems = (sz0, sz1, sz2)

    pltpu.sync_copy(idx_hbm.at[rows], idx1d)

    def _zero_body(k, _):
        zeros[pl.ds(k * 16, 16)] = jnp.zeros((16,), jnp.float32)
        return 0
    lax.fori_loop(0, VOX_PER_TILE // 16, _zero_body, 0)
    for g, sz in zip(grids, zsems):
        pltpu.async_copy(zeros, g.at[vslice], sz)
    pltpu.async_copy(feat_hbm.at[ch_lo, rows], v0, sv0)
    pltpu.async_copy(feat_hbm.at[ch_lo + 1, rows], v1, sv1)
    for g, sz in zip(grids, zsems):
        pltpu.make_async_copy(zeros, g.at[vslice], sz).wait()
    plsc.subcore_barrier()

    def _step(ch, k, zero_next):
        vk, svk, gk = vbufs[k], vsems[k], grids[k]
        n = (k + 1) % 3
        vp, svp = vbufs[(k + 2) % 3], vsems[(k + 2) % 3]
        pltpu.make_async_copy(feat_hbm.at[ch, rows], vk, svk).wait()
        pltpu.async_copy(
            feat_hbm.at[jnp.minimum(ch + 2, ch_hi), rows], vp, svp)
        if zero_next:
            pltpu.make_async_copy(grids[n].at[vslice],
                                  out_hbm.at[ch, vslice], wsems[n]).wait()
            pltpu.async_copy(zeros, grids[n].at[vslice], zsems[n])
        pltpu.sync_copy(vk, gk.at[idx1d], add=True)
        if zero_next:
            pltpu.make_async_copy(zeros, grids[n].at[vslice], zsems[n]).wait()
        plsc.subcore_barrier()
        pltpu.async_copy(gk.at[vslice], out_hbm.at[ch, vslice], wsems[k])

    _step(ch_lo + 0, 0, False)
    _step(ch_lo + 1, 1, False)

    def _trip_body(i, _):
        ch = ch_lo + 2 + 3 * i
        _step(ch + 0, 2, True)
        _step(ch + 1, 0, True)
        _step(ch + 2, 1, True)
        return 0

    lax.fori_loop(0, (CH_PER_SC - 3) // 3, _trip_body, 0)
    _step(ch_hi, 2, True)

    pltpu.make_async_copy(g1.at[vslice], out_hbm.at[ch_hi, vslice], swb1).wait()
    pltpu.make_async_copy(g2.at[vslice], out_hbm.at[ch_hi, vslice], swb2).wait()
    pltpu.make_async_copy(feat_hbm.at[ch_hi, rows], v0, sv0).wait()
    pltpu.make_async_copy(feat_hbm.at[ch_hi, rows], v1, sv1).wait()


@jax.jit
def kernel(features_2d, projection_indices):
    feat = features_2d.reshape(NCH, NPIX)
    idx = projection_indices.reshape(NPIX)

    mesh = plsc.VectorSubcoreMesh(core_axis_name="c", subcore_axis_name="s")
    run = pl.kernel(
        _sc_scatter,
        mesh=mesh,
        out_type=jax.ShapeDtypeStruct((NCH, VOX), jnp.float32),
        scratch_types=[
            pltpu.VMEM((PIX_PER_TILE,), jnp.float32),
            pltpu.VMEM((PIX_PER_TILE,), jnp.float32),
            pltpu.VMEM((PIX_PER_TILE,), jnp.float32),
            pltpu.VMEM((PIX_PER_TILE,), jnp.int32),
            pltpu.VMEM((VOX_PER_TILE,), jnp.float32),
            pltpu.VMEM_SHARED((VOX,), jnp.float32),
            pltpu.VMEM_SHARED((VOX,), jnp.float32),
            pltpu.VMEM_SHARED((VOX,), jnp.float32),
            pltpu.SemaphoreType.DMA,
            pltpu.SemaphoreType.DMA,
            pltpu.SemaphoreType.DMA,
            pltpu.SemaphoreType.DMA,
            pltpu.SemaphoreType.DMA,
            pltpu.SemaphoreType.DMA,
            pltpu.SemaphoreType.DMA,
            pltpu.SemaphoreType.DMA,
            pltpu.SemaphoreType.DMA,
        ],
    )
    out = run(feat, idx)
    return out.reshape(B, C, 128, 128, 16)


# speedup vs baseline: 1.0007x; 1.0007x over previous
import jax
import jax.numpy as jnp
from jax import lax
from jax.experimental import pallas as pl
from jax.experimental.pallas import tpu as pltpu
from jax.experimental.pallas import tpu_sc as plsc

B, C, H, W = 2, 96, 384, 384
VOX = 128 * 128 * 16
NPIX = H * W
NCH = B * C
NC, NS = 2, 16
CH_PER_SC = NCH // NC
PIX_PER_TILE = NPIX // NS
VOX_PER_TILE = VOX // NS


def _sc_scatter(feat_hbm, idx_hbm, out_hbm,
                v0, v1, v2, idx1d, zeros, g0, g1, g2,
                sv0, sv1, sv2, swb0, swb1, swb2, sz0, sz1, sz2):
    c = lax.axis_index("c")
    s = lax.axis_index("s")
    rows = pl.ds(s * PIX_PER_TILE, PIX_PER_TILE)
    ch_lo = c * CH_PER_SC
    ch_hi = ch_lo + CH_PER_SC - 1
    x0 = s * VOX_PER_TILE
    vslice = pl.ds(x0, VOX_PER_TILE)

    vbufs = (v0, v1, v2)
    vsems = (sv0, sv1, sv2)
    grids = (g0, g1, g2)
    wsems = (swb0, swb1, swb2)
    zsems = (sz0, sz1, sz2)

    pltpu.sync_copy(idx_hbm.at[rows], idx1d)

    def _zero_body(k, _):
        zeros[pl.ds(k * 16, 16)] = jnp.zeros((16,), jnp.float32)
        return 0
    lax.fori_loop(0, VOX_PER_TILE // 16, _zero_body, 0)
    for g, sz in zip(grids, zsems):
        pltpu.async_copy(zeros, g.at[vslice], sz)
    pltpu.async_copy(feat_hbm.at[ch_lo, rows], v0, sv0)
    pltpu.async_copy(feat_hbm.at[ch_lo + 1, rows], v1, sv1)

    def _step(ch, k, zero_next):
        vk, svk, gk = vbufs[k], vsems[k], grids[k]
        n = (k + 1) % 3
        vp, svp = vbufs[(k + 2) % 3], vsems[(k + 2) % 3]
        pltpu.make_async_copy(feat_hbm.at[ch, rows], vk, svk).wait()
        pltpu.async_copy(
            feat_hbm.at[jnp.minimum(ch + 2, ch_hi), rows], vp, svp)
        if zero_next:
            pltpu.make_async_copy(grids[n].at[vslice],
                                  out_hbm.at[ch, vslice], wsems[n]).wait()
            pltpu.async_copy(zeros, grids[n].at[vslice], zsems[n])
        pltpu.make_async_copy(zeros, gk.at[vslice], zsems[k]).wait()
        plsc.subcore_barrier()
        pltpu.sync_copy(vk, gk.at[idx1d], add=True)
        plsc.subcore_barrier()
        pltpu.async_copy(gk.at[vslice], out_hbm.at[ch, vslice], wsems[k])

    _step(ch_lo + 0, 0, False)
    _step(ch_lo + 1, 1, False)

    def _trip_body(i, _):
        ch = ch_lo + 2 + 3 * i
        _step(ch + 0, 2, True)
        _step(ch + 1, 0, True)
        _step(ch + 2, 1, True)
        return 0

    lax.fori_loop(0, (CH_PER_SC - 3) // 3, _trip_body, 0)
    _step(ch_hi, 2, True)

    pltpu.make_async_copy(g1.at[vslice], out_hbm.at[ch_hi, vslice], swb1).wait()
    pltpu.make_async_copy(g2.at[vslice], out_hbm.at[ch_hi, vslice], swb2).wait()
    pltpu.make_async_copy(zeros, g0.at[vslice], sz0).wait()
    pltpu.make_async_copy(feat_hbm.at[ch_hi, rows], v0, sv0).wait()
    pltpu.make_async_copy(feat_hbm.at[ch_hi, rows], v1, sv1).wait()


@jax.jit
def kernel(features_2d, projection_indices):
    feat = features_2d.reshape(NCH, NPIX)
    idx = projection_indices.reshape(NPIX)

    mesh = plsc.VectorSubcoreMesh(core_axis_name="c", subcore_axis_name="s")
    run = pl.kernel(
        _sc_scatter,
        mesh=mesh,
        out_type=jax.ShapeDtypeStruct((NCH, VOX), jnp.float32),
        scratch_types=[
            pltpu.VMEM((PIX_PER_TILE,), jnp.float32),
            pltpu.VMEM((PIX_PER_TILE,), jnp.float32),
            pltpu.VMEM((PIX_PER_TILE,), jnp.float32),
            pltpu.VMEM((PIX_PER_TILE,), jnp.int32),
            pltpu.VMEM((VOX_PER_TILE,), jnp.float32),
            pltpu.VMEM_SHARED((VOX,), jnp.float32),
            pltpu.VMEM_SHARED((VOX,), jnp.float32),
            pltpu.VMEM_SHARED((VOX,), jnp.float32),
            pltpu.SemaphoreType.DMA,
            pltpu.SemaphoreType.DMA,
            pltpu.SemaphoreType.DMA,
            pltpu.SemaphoreType.DMA,
            pltpu.SemaphoreType.DMA,
            pltpu.SemaphoreType.DMA,
            pltpu.SemaphoreType.DMA,
            pltpu.SemaphoreType.DMA,
            pltpu.SemaphoreType.DMA,
        ],
    )
    out = run(feat, idx)
    return out.reshape(B, C, 128, 128, 16)
